# initial kernel scaffold (unmeasured)
import functools

import jax
import jax.numpy as jnp
from jax import lax
from jax.experimental import pallas as pl
from jax.experimental.pallas import tpu as pltpu

NDEV = 16
S = 256
BB = 4
D = 256
N = 16

CDT = jnp.float32


def kernel(x, A, B, C):
    def body(x_ref, a_ref, b_ref, c_ref, y_ref, comm_ref, send_sem, recv_sem):
        my = lax.axis_index("i")
        left = lax.rem(my + NDEV - 1, NDEV)
        right = lax.rem(my + 1, NDEV)

        barrier = pltpu.get_barrier_semaphore()
        for nbr in (left, right):
            pl.semaphore_signal(
                barrier, inc=1, device_id=(nbr,),
                device_id_type=pl.DeviceIdType.MESH,
            )
        pl.semaphore_wait(barrier, 2)

        AT = a_ref[...].T
        X = x_ref[...]
        Bv = b_ref[...]
        Cv = c_ref[...]

        Xe = X.transpose(1, 0, 2)[:, :, None, :]
        Be = Bv.transpose(1, 0, 2)[:, :, :, None]
        U = (Xe * Be).astype(CDT)

        tio = lax.broadcasted_iota(jnp.float32, (S, N, D), 0)
        W = jnp.exp(AT[None] * (float(S - 1) - tio))
        h_fin = (U * W[:, None]).sum(axis=0)

        dAP = jnp.exp(AT * float(S))

        @pl.when(my == 0)
        def _():
            comm_ref[0, :, :, :] = jnp.zeros((BB, N, D), jnp.float32)

        @pl.when(my > 0)
        def _():
            recv = pltpu.make_async_remote_copy(
                src_ref=comm_ref.at[1],
                dst_ref=comm_ref.at[0],
                send_sem=send_sem,
                recv_sem=recv_sem,
                device_id=(left,),
                device_id_type=pl.DeviceIdType.MESH,
            )
            recv.wait_recv()

        h_in = comm_ref[0, :, :, :]

        @pl.when(my < NDEV - 1)
        def _():
            comm_ref[1, :, :, :] = h_in * dAP[None] + h_fin
            send = pltpu.make_async_remote_copy(
                src_ref=comm_ref.at[1],
                dst_ref=comm_ref.at[0],
                send_sem=send_sem,
                recv_sem=recv_sem,
                device_id=(right,),
                device_id_type=pl.DeviceIdType.MESH,
            )
            send.start()
            send.wait_send()

        H = U
        step = 1
        while step < S:
            dApw = jnp.exp(AT * float(step)).astype(CDT)
            Hs = jnp.concatenate(
                [jnp.zeros((step, BB, N, D), CDT), H[:-step]], axis=0
            )
            H = H + dApw[None, None] * Hs
            step *= 2

        dApF = jnp.exp(AT[None] * (tio + 1.0)).astype(CDT)
        H = H + dApF[:, None] * h_in.astype(CDT)[None]

        Ce = Cv.transpose(1, 0, 2)[:, :, :, None]
        Y = (H * Ce.astype(CDT)).sum(axis=2).astype(jnp.float32)
        for b in range(BB):
            y_ref[b, :, :] = Y[:, b, :]

        @functools.partial(pl.run_scoped, sem2=pltpu.SemaphoreType.REGULAR)
        def _(sem2):
            for nbr in (left, right):
                pl.semaphore_signal(
                    sem2, inc=1, device_id=(nbr,),
                    device_id_type=pl.DeviceIdType.MESH,
                )
            pl.semaphore_wait(sem2, 2)

    return pl.pallas_call(
        body,
        out_shape=jax.ShapeDtypeStruct((BB, S, D), jnp.float32),
        in_specs=[pl.BlockSpec(memory_space=pltpu.VMEM)] * 4,
        out_specs=pl.BlockSpec(memory_space=pltpu.VMEM),
        scratch_shapes=[
            pltpu.VMEM((2, BB, N, D), jnp.float32),
            pltpu.SemaphoreType.DMA,
            pltpu.SemaphoreType.DMA,
        ],
        compiler_params=pltpu.CompilerParams(collective_id=0),
    )(x, A, B, C)


# baseline (device time: 64635 ns/iter reference)
import functools

import jax
import jax.numpy as jnp
from jax import lax
from jax.experimental import pallas as pl
from jax.experimental.pallas import tpu as pltpu

NDEV = 16
S = 256
BB = 4
D = 256
N = 16

CDT = jnp.float32


def kernel(x, A, B, C):
    def body(x_ref, a_ref, b_ref, c_ref, y_ref, comm_ref, send_sem, recv_sem):
        my = lax.axis_index("i")
        left = lax.rem(my + NDEV - 1, NDEV)
        right = lax.rem(my + 1, NDEV)

        barrier = pltpu.get_barrier_semaphore()
        for nbr in (left, right):
            pl.semaphore_signal(
                barrier, inc=1, device_id=(nbr,),
                device_id_type=pl.DeviceIdType.MESH,
            )
        pl.semaphore_wait(barrier, 2)

        AT = a_ref[...].T
        X = x_ref[...]
        Bv = b_ref[...]
        Cv = c_ref[...]

        Xe = X.transpose(1, 0, 2)[:, :, None, :]
        Be = Bv.transpose(1, 0, 2)[:, :, :, None]
        U = (Xe * Be).astype(CDT)

        tio = lax.broadcasted_iota(jnp.int32, (S, N, D), 0).astype(jnp.float32)
        W = jnp.exp(AT[None] * (float(S - 1) - tio))
        h_fin = (U * W[:, None]).sum(axis=0)

        dAP = jnp.exp(AT * float(S))

        @pl.when(my == 0)
        def _():
            comm_ref[0, :, :, :] = jnp.zeros((BB, N, D), jnp.float32)

        @pl.when(my > 0)
        def _():
            recv = pltpu.make_async_remote_copy(
                src_ref=comm_ref.at[1],
                dst_ref=comm_ref.at[0],
                send_sem=send_sem,
                recv_sem=recv_sem,
                device_id=(left,),
                device_id_type=pl.DeviceIdType.MESH,
            )
            recv.wait_recv()

        h_in = comm_ref[0, :, :, :]

        @pl.when(my < NDEV - 1)
        def _():
            comm_ref[1, :, :, :] = h_in * dAP[None] + h_fin
            send = pltpu.make_async_remote_copy(
                src_ref=comm_ref.at[1],
                dst_ref=comm_ref.at[0],
                send_sem=send_sem,
                recv_sem=recv_sem,
                device_id=(right,),
                device_id_type=pl.DeviceIdType.MESH,
            )
            send.start()
            send.wait_send()

        H = U
        step = 1
        while step < S:
            dApw = jnp.exp(AT * float(step)).astype(CDT)
            Hs = jnp.concatenate(
                [jnp.zeros((step, BB, N, D), CDT), H[:-step]], axis=0
            )
            H = H + dApw[None, None] * Hs
            step *= 2

        dApF = jnp.exp(AT[None] * (tio + 1.0)).astype(CDT)
        H = H + dApF[:, None] * h_in.astype(CDT)[None]

        Ce = Cv.transpose(1, 0, 2)[:, :, :, None]
        Y = (H * Ce.astype(CDT)).sum(axis=2).astype(jnp.float32)
        for b in range(BB):
            y_ref[b, :, :] = Y[:, b, :]

        @functools.partial(pl.run_scoped, sem2=pltpu.SemaphoreType.REGULAR)
        def _(sem2):
            for nbr in (left, right):
                pl.semaphore_signal(
                    sem2, inc=1, device_id=(nbr,),
                    device_id_type=pl.DeviceIdType.MESH,
                )
            pl.semaphore_wait(sem2, 2)

    return pl.pallas_call(
        body,
        out_shape=jax.ShapeDtypeStruct((BB, S, D), jnp.float32),
        in_specs=[pl.BlockSpec(memory_space=pltpu.VMEM)] * 4,
        out_specs=pl.BlockSpec(memory_space=pltpu.VMEM),
        scratch_shapes=[
            pltpu.VMEM((2, BB, N, D), jnp.float32),
            pltpu.SemaphoreType.DMA,
            pltpu.SemaphoreType.DMA,
        ],
        compiler_params=pltpu.CompilerParams(collective_id=0),
    )(x, A, B, C)
